# baseline (speedup 1.0000x reference)
import functools

import jax
import jax.numpy as jnp
from jax import lax
from jax.experimental import pallas as pl
from jax.experimental.pallas import tpu as pltpu
from jax.experimental.pallas import tpu_sc as plsc

_GATHER_WINDOW = 128


_NUM_SC_WORKERS = 32


def _sc_gather(emb2d, flat_idx, start, count):
    d = emb2d.shape[1]
    w = _GATHER_WINDOW
    nbuf = 4
    n = count // (w * _NUM_SC_WORKERS)
    assert n * w * _NUM_SC_WORKERS == count and n >= nbuf
    mesh = plsc.VectorSubcoreMesh(core_axis_name="core", subcore_axis_name="subcore")

    @functools.partial(
        pl.kernel,
        out_type=jax.ShapeDtypeStruct((count, d), emb2d.dtype),
        mesh=mesh,
        scratch_types=[
            pltpu.VMEM((n * w,), jnp.int32),
        ]
        + [pltpu.VMEM((w, d), emb2d.dtype) for _ in range(nbuf)]
        + [pltpu.SemaphoreType.DMA for _ in range(2 * nbuf)],
    )
    def gather_kernel(x_hbm, i_hbm, o_hbm, idx_v, *bufs_sems):
        bufs = bufs_sems[:nbuf]
        gsems = bufs_sems[nbuf:2 * nbuf]
        wsems = bufs_sems[2 * nbuf:]
        wid = lax.axis_index("subcore") * 2 + lax.axis_index("core")
        row0 = wid * n * w
        pltpu.sync_copy(i_hbm.at[pl.ds(start + row0, n * w)], idx_v)

        def start_gather(win, j):
            pltpu.async_copy(
                x_hbm.at[idx_v.at[pl.ds(win * w, w)]], bufs[j], gsems[j]
            )

        def wait_gather(j):
            pltpu.make_async_copy(
                x_hbm.at[idx_v.at[pl.ds(0, w)]], bufs[j], gsems[j]
            ).wait()

        def start_write(win, j):
            pltpu.async_copy(
                bufs[j], o_hbm.at[pl.ds(row0 + win * w, w), :], wsems[j]
            )

        def wait_write(j):
            pltpu.make_async_copy(
                bufs[j], o_hbm.at[pl.ds(row0, w), :], wsems[j]
            ).wait()

        for win in range(nbuf):
            start_gather(win, win)
        for win in range(n):
            j = win % nbuf
            wait_gather(j)
            start_write(win, j)
            nxt = win + nbuf
            if nxt < n:
                wait_write(j)
                start_gather(nxt, j)
        for win in range(max(0, n - nbuf), n):
            wait_write(win % nbuf)

    return gather_kernel(emb2d, flat_idx)


def _decode_body(rb, k, cand_ref, w1_ref, b1_ref, w2_ref, b2_ref, w3_ref,
                 b3_ref, mask_ref, gum_ref, logp_ref, act_ref):
    x = cand_ref[...]
    h = jnp.tanh(jnp.dot(x, w1_ref[...]) + b1_ref[...])
    h = jnp.tanh(jnp.dot(h, w2_ref[...]) + b2_ref[...])
    logits = jnp.dot(h, w3_ref[...]) + b3_ref[...]
    logits = logits.reshape(rb, k)
    mask = mask_ref[...]
    neg_inf = jnp.float32(-jnp.inf)
    logits = jnp.where(mask, logits, neg_inf)
    xm = jnp.max(logits, axis=1, keepdims=True)
    shifted = logits - xm
    lse = jnp.log(jnp.sum(jnp.exp(shifted), axis=1, keepdims=True))
    logp = shifted - lse
    logp_ref[...] = logp
    gumbel = -jnp.log(-jnp.log(gum_ref[...]))
    keys = jnp.where(mask, logp + gumbel, neg_inf)
    km = jnp.max(keys, axis=1, keepdims=True)
    iota = lax.broadcasted_iota(jnp.int32, (rb, k), 1)
    first_max = jnp.min(jnp.where(keys == km, iota, k), axis=1, keepdims=True)
    act_ref[...] = jnp.broadcast_to(first_max, act_ref.shape)


def _tc_decode(cand2d, W1, b1, W2, b2, W3, b3, action_mask, gumbel_u,
               rb, row0, rows):
    k = action_mask.shape[1]
    d = cand2d.shape[1]
    grid = (rows // rb,)
    blk0 = row0 // rb
    body = functools.partial(_decode_body, rb, k)
    return pl.pallas_call(
        body,
        grid=grid,
        in_specs=[
            pl.BlockSpec((rb * k, d), lambda i: (i, 0)),
            pl.BlockSpec((d, d), lambda i: (0, 0)),
            pl.BlockSpec((1, d), lambda i: (0, 0)),
            pl.BlockSpec((d, d), lambda i: (0, 0)),
            pl.BlockSpec((1, d), lambda i: (0, 0)),
            pl.BlockSpec((d, 1), lambda i: (0, 0)),
            pl.BlockSpec((1, 1), lambda i: (0, 0)),
            pl.BlockSpec((rb, k), lambda i: (i + blk0, 0)),
            pl.BlockSpec((rb, k), lambda i: (i + blk0, 0)),
        ],
        out_specs=[
            pl.BlockSpec((rb, k), lambda i: (i, 0)),
            pl.BlockSpec((rb, 128), lambda i: (i, 0)),
        ],
        out_shape=[
            jax.ShapeDtypeStruct((rows, k), jnp.float32),
            jax.ShapeDtypeStruct((rows, 128), jnp.int32),
        ],
        compiler_params=pltpu.CompilerParams(
            dimension_semantics=("parallel",),
        ),
    )(cand2d, W1, b1.reshape(1, d), W2, b2.reshape(1, d), W3,
      b3.reshape(1, 1), action_mask, gumbel_u)


def kernel(embeddings, gumbel_u, W1, b1, W2, b2, W3, b3, next_op, action_mask):
    b, n, d = embeddings.shape
    k = next_op.shape[1]
    emb2d = embeddings.reshape(b * n, d)
    flat_idx = (
        next_op.astype(jnp.int32)
        + (jnp.arange(b, dtype=jnp.int32) * n)[:, None]
    ).reshape(b * k)
    chunk_rows = (48, 48, 32)
    logps, acts = [], []
    row0 = 0
    for rows in chunk_rows:
        cand_c = _sc_gather(emb2d, flat_idx, row0 * k, rows * k)
        lp, ac = _tc_decode(
            cand_c, W1, b1, W2, b2, W3, b3,
            action_mask, gumbel_u, rb=8, row0=row0, rows=rows,
        )
        logps.append(lp)
        acts.append(ac)
        row0 += rows
    log_p = jnp.concatenate(logps, axis=0)
    actions = jnp.concatenate(acts, axis=0)[:, 0]
    return (log_p, actions)


# speedup vs baseline: 1.0295x; 1.0295x over previous
import functools

import jax
import jax.numpy as jnp
from jax import lax
from jax.experimental import pallas as pl
from jax.experimental.pallas import tpu as pltpu
from jax.experimental.pallas import tpu_sc as plsc

_GATHER_WINDOW = 128


_NUM_SC_WORKERS = 32


def _sc_gather(emb2d, flat_idx, start, count):
    d = emb2d.shape[1]
    w = _GATHER_WINDOW
    nbuf = 4
    n = count // (w * _NUM_SC_WORKERS)
    assert n * w * _NUM_SC_WORKERS == count and n >= nbuf
    mesh = plsc.VectorSubcoreMesh(core_axis_name="core", subcore_axis_name="subcore")

    @functools.partial(
        pl.kernel,
        out_type=jax.ShapeDtypeStruct((count, d), emb2d.dtype),
        mesh=mesh,
        scratch_types=[
            pltpu.VMEM((n * w,), jnp.int32),
        ]
        + [pltpu.VMEM((w, d), emb2d.dtype) for _ in range(nbuf)]
        + [pltpu.SemaphoreType.DMA for _ in range(2 * nbuf)],
    )
    def gather_kernel(x_hbm, i_hbm, o_hbm, idx_v, *bufs_sems):
        bufs = bufs_sems[:nbuf]
        gsems = bufs_sems[nbuf:2 * nbuf]
        wsems = bufs_sems[2 * nbuf:]
        wid = lax.axis_index("subcore") * 2 + lax.axis_index("core")
        row0 = wid * n * w
        pltpu.sync_copy(i_hbm.at[pl.ds(start + row0, n * w)], idx_v)

        def start_gather(win, j):
            pltpu.async_copy(
                x_hbm.at[idx_v.at[pl.ds(win * w, w)]], bufs[j], gsems[j]
            )

        def wait_gather(j):
            pltpu.make_async_copy(
                x_hbm.at[idx_v.at[pl.ds(0, w)]], bufs[j], gsems[j]
            ).wait()

        def start_write(win, j):
            pltpu.async_copy(
                bufs[j], o_hbm.at[pl.ds(row0 + win * w, w), :], wsems[j]
            )

        def wait_write(j):
            pltpu.make_async_copy(
                bufs[j], o_hbm.at[pl.ds(row0, w), :], wsems[j]
            ).wait()

        for win in range(nbuf):
            start_gather(win, win)
        for win in range(n):
            j = win % nbuf
            wait_gather(j)
            start_write(win, j)
            nxt = win + nbuf
            if nxt < n:
                wait_write(j)
                start_gather(nxt, j)
        for win in range(max(0, n - nbuf), n):
            wait_write(win % nbuf)

    return gather_kernel(emb2d, flat_idx)


def _decode_body(rb, k, cand_ref, w1_ref, b1_ref, w2_ref, b2_ref, w3_ref,
                 b3_ref, mask_ref, gum_ref, logp_ref, act_ref):
    x = cand_ref[...]
    h = jnp.tanh(jnp.dot(x, w1_ref[...]) + b1_ref[...])
    h = jnp.tanh(jnp.dot(h, w2_ref[...]) + b2_ref[...])
    logits = jnp.dot(h, w3_ref[...]) + b3_ref[...]
    logits = logits.reshape(rb, k)
    mask = mask_ref[...]
    neg_inf = jnp.float32(-jnp.inf)
    logits = jnp.where(mask, logits, neg_inf)
    xm = jnp.max(logits, axis=1, keepdims=True)
    shifted = logits - xm
    lse = jnp.log(jnp.sum(jnp.exp(shifted), axis=1, keepdims=True))
    logp = shifted - lse
    logp_ref[...] = logp
    gumbel = -jnp.log(-jnp.log(gum_ref[...]))
    keys = jnp.where(mask, logp + gumbel, neg_inf)
    km = jnp.max(keys, axis=1, keepdims=True)
    iota = lax.broadcasted_iota(jnp.int32, (rb, k), 1)
    first_max = jnp.min(jnp.where(keys == km, iota, k), axis=1, keepdims=True)
    act_ref[...] = jnp.broadcast_to(first_max, act_ref.shape)


def _tc_decode(cand2d, W1, b1, W2, b2, W3, b3, action_mask, gumbel_u,
               rb, row0, rows):
    k = action_mask.shape[1]
    d = cand2d.shape[1]
    grid = (rows // rb,)
    blk0 = row0 // rb
    body = functools.partial(_decode_body, rb, k)
    return pl.pallas_call(
        body,
        grid=grid,
        in_specs=[
            pl.BlockSpec((rb * k, d), lambda i: (i, 0)),
            pl.BlockSpec((d, d), lambda i: (0, 0)),
            pl.BlockSpec((1, d), lambda i: (0, 0)),
            pl.BlockSpec((d, d), lambda i: (0, 0)),
            pl.BlockSpec((1, d), lambda i: (0, 0)),
            pl.BlockSpec((d, 1), lambda i: (0, 0)),
            pl.BlockSpec((1, 1), lambda i: (0, 0)),
            pl.BlockSpec((rb, k), lambda i: (i + blk0, 0)),
            pl.BlockSpec((rb, k), lambda i: (i + blk0, 0)),
        ],
        out_specs=[
            pl.BlockSpec((rb, k), lambda i: (i, 0)),
            pl.BlockSpec((rb, 128), lambda i: (i, 0)),
        ],
        out_shape=[
            jax.ShapeDtypeStruct((rows, k), jnp.float32),
            jax.ShapeDtypeStruct((rows, 128), jnp.int32),
        ],
        compiler_params=pltpu.CompilerParams(
            dimension_semantics=("parallel",),
        ),
    )(cand2d, W1, b1.reshape(1, d), W2, b2.reshape(1, d), W3,
      b3.reshape(1, 1), action_mask, gumbel_u)


def kernel(embeddings, gumbel_u, W1, b1, W2, b2, W3, b3, next_op, action_mask):
    b, n, d = embeddings.shape
    k = next_op.shape[1]
    emb2d = embeddings.reshape(b * n, d)
    flat_idx = (
        next_op.astype(jnp.int32)
        + (jnp.arange(b, dtype=jnp.int32) * n)[:, None]
    ).reshape(b * k)
    chunk_rows = (32, 32, 32, 32)
    logps, acts = [], []
    row0 = 0
    for rows in chunk_rows:
        cand_c = _sc_gather(emb2d, flat_idx, row0 * k, rows * k)
        lp, ac = _tc_decode(
            cand_c, W1, b1, W2, b2, W3, b3,
            action_mask, gumbel_u, rb=16, row0=row0, rows=rows,
        )
        logps.append(lp)
        acts.append(ac)
        row0 += rows
    log_p = jnp.concatenate(logps, axis=0)
    actions = jnp.concatenate(acts, axis=0)[:, 0]
    return (log_p, actions)


# speedup vs baseline: 1.0336x; 1.0040x over previous
import functools

import jax
import jax.numpy as jnp
from jax import lax
from jax.experimental import pallas as pl
from jax.experimental.pallas import tpu as pltpu
from jax.experimental.pallas import tpu_sc as plsc

_GATHER_WINDOW = 128


_NUM_SC_WORKERS = 32


def _sc_gather(emb2d, flat_idx, start, count):
    d = emb2d.shape[1]
    w = _GATHER_WINDOW
    nbuf = 4
    n = count // (w * _NUM_SC_WORKERS)
    assert n * w * _NUM_SC_WORKERS == count and n >= nbuf
    mesh = plsc.VectorSubcoreMesh(core_axis_name="core", subcore_axis_name="subcore")

    @functools.partial(
        pl.kernel,
        out_type=jax.ShapeDtypeStruct((count, d), emb2d.dtype),
        mesh=mesh,
        scratch_types=[
            pltpu.VMEM((n * w,), jnp.int32),
        ]
        + [pltpu.VMEM((w, d), emb2d.dtype) for _ in range(nbuf)]
        + [pltpu.SemaphoreType.DMA for _ in range(2 * nbuf)],
    )
    def gather_kernel(x_hbm, i_hbm, o_hbm, idx_v, *bufs_sems):
        bufs = bufs_sems[:nbuf]
        gsems = bufs_sems[nbuf:2 * nbuf]
        wsems = bufs_sems[2 * nbuf:]
        wid = lax.axis_index("subcore") * 2 + lax.axis_index("core")
        row0 = wid * n * w
        pltpu.sync_copy(i_hbm.at[pl.ds(start + row0, n * w)], idx_v)

        def start_gather(win, j):
            pltpu.async_copy(
                x_hbm.at[idx_v.at[pl.ds(win * w, w)]], bufs[j], gsems[j]
            )

        def wait_gather(j):
            pltpu.make_async_copy(
                x_hbm.at[idx_v.at[pl.ds(0, w)]], bufs[j], gsems[j]
            ).wait()

        def start_write(win, j):
            pltpu.async_copy(
                bufs[j], o_hbm.at[pl.ds(row0 + win * w, w), :], wsems[j]
            )

        def wait_write(j):
            pltpu.make_async_copy(
                bufs[j], o_hbm.at[pl.ds(row0, w), :], wsems[j]
            ).wait()

        for win in range(nbuf):
            start_gather(win, win)
        for win in range(n):
            j = win % nbuf
            wait_gather(j)
            start_write(win, j)
            nxt = win + nbuf
            if nxt < n:
                wait_write(j)
                start_gather(nxt, j)
        for win in range(max(0, n - nbuf), n):
            wait_write(win % nbuf)

    return gather_kernel(emb2d, flat_idx)


def _decode_body(rb, k, cand_ref, w1_ref, b1_ref, w2_ref, b2_ref, w3_ref,
                 b3_ref, mask_ref, gum_ref, logp_ref, act_ref):
    x = cand_ref[...]
    h = jnp.tanh(jnp.dot(x, w1_ref[...]) + b1_ref[...])
    h = jnp.tanh(jnp.dot(h, w2_ref[...]) + b2_ref[...])
    logits = jnp.dot(h, w3_ref[...]) + b3_ref[...]
    logits = logits.reshape(rb, k)
    mask = mask_ref[...]
    neg_inf = jnp.float32(-jnp.inf)
    logits = jnp.where(mask, logits, neg_inf)
    xm = jnp.max(logits, axis=1, keepdims=True)
    shifted = logits - xm
    lse = jnp.log(jnp.sum(jnp.exp(shifted), axis=1, keepdims=True))
    logp = shifted - lse
    logp_ref[...] = logp
    gumbel = -jnp.log(-jnp.log(gum_ref[...]))
    keys = jnp.where(mask, logp + gumbel, neg_inf)
    km = jnp.max(keys, axis=1, keepdims=True)
    iota = lax.broadcasted_iota(jnp.int32, (rb, k), 1)
    first_max = jnp.min(jnp.where(keys == km, iota, k), axis=1, keepdims=True)
    act_ref[...] = jnp.broadcast_to(first_max, act_ref.shape)


def _tc_decode(cand2d, W1, b1, W2, b2, W3, b3, action_mask, gumbel_u,
               rb, row0, rows):
    k = action_mask.shape[1]
    d = cand2d.shape[1]
    grid = (rows // rb,)
    blk0 = row0 // rb
    body = functools.partial(_decode_body, rb, k)
    return pl.pallas_call(
        body,
        grid=grid,
        in_specs=[
            pl.BlockSpec((rb * k, d), lambda i: (i, 0)),
            pl.BlockSpec((d, d), lambda i: (0, 0)),
            pl.BlockSpec((1, d), lambda i: (0, 0)),
            pl.BlockSpec((d, d), lambda i: (0, 0)),
            pl.BlockSpec((1, d), lambda i: (0, 0)),
            pl.BlockSpec((d, 1), lambda i: (0, 0)),
            pl.BlockSpec((1, 1), lambda i: (0, 0)),
            pl.BlockSpec((rb, k), lambda i: (i + blk0, 0)),
            pl.BlockSpec((rb, k), lambda i: (i + blk0, 0)),
        ],
        out_specs=[
            pl.BlockSpec((rb, k), lambda i: (i, 0)),
            pl.BlockSpec((rb, 128), lambda i: (i, 0)),
        ],
        out_shape=[
            jax.ShapeDtypeStruct((rows, k), jnp.float32),
            jax.ShapeDtypeStruct((rows, 128), jnp.int32),
        ],
        compiler_params=pltpu.CompilerParams(
            dimension_semantics=("parallel",),
        ),
    )(cand2d, W1, b1.reshape(1, d), W2, b2.reshape(1, d), W3,
      b3.reshape(1, 1), action_mask, gumbel_u)


def kernel(embeddings, gumbel_u, W1, b1, W2, b2, W3, b3, next_op, action_mask):
    b, n, d = embeddings.shape
    k = next_op.shape[1]
    emb2d = embeddings.reshape(b * n, d)
    flat_idx = (
        next_op.astype(jnp.int32)
        + (jnp.arange(b, dtype=jnp.int32) * n)[:, None]
    ).reshape(b * k)
    chunk_rows = (32, 32, 32, 32)
    logps, acts = [], []
    row0 = 0
    for rows in chunk_rows:
        cand_c = _sc_gather(emb2d, flat_idx, row0 * k, rows * k)
        lp, ac = _tc_decode(
            cand_c, W1, b1, W2, b2, W3, b3,
            action_mask, gumbel_u, rb=16, row0=row0, rows=rows,
        )
        logps.append(lp)
        acts.append(ac)
        row0 += rows
    log_p = jnp.zeros((b, k), jnp.float32)
    row0 = 0
    for rows, lp in zip(chunk_rows, logps):
        log_p = lax.dynamic_update_slice(log_p, lp, (row0, 0))
        row0 += rows
    actions = jnp.concatenate(acts, axis=0)[:, 0]
    return (log_p, actions)
